# Initial kernel scaffold; baseline (speedup 1.0000x reference)
import jax
import jax.numpy as jnp
from jax.experimental import pallas as pl


def kernel(dst_feat, src_feat_a, src_feat_b, edge_index_a, edge_index_b, W_T_dst, b_T_dst, W_T_a, b_T_a, gatA_W, gatA_attn_l, gatA_attn_r, gatA_bias, gatB_W, gatB_attn_l, gatB_attn_r, gatB_bias, sem_W1, sem_b1, sem_W2):
    raise NotImplementedError("write your pallas kernel here")


# speedup vs baseline: 13.2986x; 13.2986x over previous
import functools

import jax
import jax.numpy as jnp
from jax import lax
from jax.experimental import pallas as pl
from jax.experimental.pallas import tpu as pltpu
from jax.experimental.pallas import tpu_sc as plsc

N = 10000
E = 160000
D_RAW = 128
D = 256
H_SEM = 128

NC = 2
NS = 16
EPT = E // NS
ROWS_PT = 640
NPAD = NS * ROWS_PT
SEG = 2000
NSEG = EPT // SEG
BLK = 80
NBLK = SEG // BLK
DEN_W = 16


_BN1 = 1024
_NS1 = 10240


def _stage1_body(dst_ref, sa_ref, sb_ref, wtd_ref, btd_ref, wta_ref, bta_ref,
                 wga_ref, ala_ref, ara_ref, wgb_ref, alb_ref, arb_ref,
                 fa_ref, fb_ref, scal_ref):
    f32 = jnp.float32
    dst = dst_ref[...]
    h_dst = jnp.dot(dst, wtd_ref[...], preferred_element_type=f32) + btd_ref[...][None, :]
    h_a = jnp.dot(sa_ref[...], wta_ref[...], preferred_element_type=f32) + bta_ref[...][None, :]
    feat_a = jnp.dot(h_a, wga_ref[...], preferred_element_type=f32)
    feat_b = jnp.dot(sb_ref[...], wgb_ref[...], preferred_element_type=f32)
    el_a = jnp.dot(feat_a, ala_ref[...], preferred_element_type=f32)
    el_b = jnp.dot(feat_b, alb_ref[...], preferred_element_type=f32)
    v_a = jnp.dot(wga_ref[...], ara_ref[...], preferred_element_type=f32)
    v_b = jnp.dot(wgb_ref[...], arb_ref[...], preferred_element_type=f32)
    er_a = jnp.dot(h_dst, v_a, preferred_element_type=f32)
    er_b = jnp.dot(h_dst, v_b, preferred_element_type=f32)
    fa_ref[0] = feat_a[:, :128]
    fa_ref[1] = feat_a[:, 128:]
    fb_ref[0] = feat_b[:, :128]
    fb_ref[1] = feat_b[:, 128:]
    zero = jnp.zeros((4, _BN1), jnp.float32)
    scal_ref[...] = jnp.concatenate(
        [el_a[None, :], er_a[None, :], el_b[None, :], er_b[None, :], zero], axis=0)


def _stage1(dst_feat, src_feat_a, src_feat_b, W_T_dst, b_T_dst, W_T_a, b_T_a,
            gatA_W, gatA_attn_l, gatA_attn_r, gatB_W, gatB_attn_l, gatB_attn_r):
    f32 = jnp.float32
    grid = _NS1 // _BN1
    full = lambda shape: pl.BlockSpec(shape, lambda i: tuple(0 for _ in shape))
    return pl.pallas_call(
        _stage1_body,
        grid=(grid,),
        in_specs=[
            pl.BlockSpec((_BN1, D_RAW), lambda i: (i, 0)),
            pl.BlockSpec((_BN1, D_RAW), lambda i: (i, 0)),
            pl.BlockSpec((_BN1, D), lambda i: (i, 0)),
            full((D_RAW, D)), full((D,)), full((D_RAW, D)), full((D,)),
            full((D, D)), full((D,)), full((D,)),
            full((D, D)), full((D,)), full((D,)),
        ],
        out_specs=[
            pl.BlockSpec((2, _BN1, 128), lambda i: (0, i, 0)),
            pl.BlockSpec((2, _BN1, 128), lambda i: (0, i, 0)),
            pl.BlockSpec((8, _BN1), lambda i: (0, i)),
        ],
        out_shape=[
            jax.ShapeDtypeStruct((2, N, 128), f32),
            jax.ShapeDtypeStruct((2, N, 128), f32),
            jax.ShapeDtypeStruct((8, _NS1), f32),
        ],
    )(dst_feat, src_feat_a, src_feat_b, W_T_dst, b_T_dst, W_T_a, b_T_a,
      gatA_W, gatA_attn_l, gatA_attn_r, gatB_W, gatB_attn_l, gatB_attn_r)


def _sc_body(scal_hbm, feat_a_hbm, feat_b_hbm, src_a_hbm, dst_a_hbm,
             src_b_hbm, dst_b_hbm,
             outf_a_hbm, outd_a_hbm, outf_b_hbm, outd_b_hbm,
             el_v, er_v, srcb, dstb, wblk, rixblk, gbuf, wsplat, dstidx, sem,
             acc, accd):
    f32 = jnp.float32
    c = lax.axis_index("c")
    s = lax.axis_index("s")
    cN = (c * N).astype(jnp.int32)
    base_e = s * EPT
    row0 = s * ROWS_PT

    z16 = jnp.zeros((16,), f32)

    def zero_bufs(g, _):
        for j in range(8):
            gbuf[g, pl.ds(j * 16, 16)] = z16
        wsplat[g, pl.ds(0, 16)] = z16
        return 0

    def zero_acc():
        lax.fori_loop(0, BLK, zero_bufs, 0)
        for k in range(ROWS_PT // BLK):
            pltpu.sync_copy(gbuf, acc.at[pl.ds(row0 + k * BLK, BLK)])
            pltpu.sync_copy(wsplat, accd.at[pl.ds(row0 + k * BLK, BLK)])
        plsc.subcore_barrier()

    zero_acc()

    for rel in range(2):
        feat_hbm = feat_a_hbm if rel == 0 else feat_b_hbm
        src_hbm = src_a_hbm if rel == 0 else src_b_hbm
        dst_hbm = dst_a_hbm if rel == 0 else dst_b_hbm
        outf_hbm = outf_a_hbm if rel == 0 else outf_b_hbm
        outd_hbm = outd_a_hbm if rel == 0 else outd_b_hbm

        pltpu.sync_copy(scal_hbm.at[2 * rel], el_v)
        pltpu.sync_copy(scal_hbm.at[2 * rel + 1], er_v)

        for seg in range(NSEG):
            soff = base_e + seg * SEG
            pltpu.sync_copy(src_hbm.at[pl.ds(soff, SEG)], srcb)
            pltpu.sync_copy(dst_hbm.at[pl.ds(soff, SEG)], dstb)

            def bbody(b, _):
                boff = b * BLK
                for j in range(BLK // 16):
                    s16 = srcb[pl.ds(boff + j * 16, 16)]
                    d16 = dstb[pl.ds(boff + j * 16, 16)]
                    e = (plsc.load_gather(el_v, [s16])
                         + plsc.load_gather(er_v, [d16]))
                    e = jnp.where(e > 0, e, 0.2 * e)
                    wblk[pl.ds(j * 16, 16)] = jnp.exp(e)
                    rixblk[pl.ds(j * 16, 16)] = s16 + cN
                    dstidx[0, pl.ds(j * 16, 16)] = d16
                pltpu.async_copy(feat_hbm.at[rixblk], gbuf, sem).wait()

                def rbody(gi, _):
                    w16 = wblk[pl.ds(gi * 16, 16)]
                    g0 = gi * 16
                    for l in range(16):
                        g = g0 + l
                        wg = w16[l]
                        for j in range(8):
                            gbuf[g, pl.ds(j * 16, 16)] = gbuf[g, pl.ds(j * 16, 16)] * wg
                        wsplat[g, pl.ds(0, 16)] = jnp.full((16,), wg, f32)
                    return 0

                lax.fori_loop(0, BLK // 16, rbody, 0)
                pltpu.sync_copy(gbuf, acc.at[dstidx.at[0]], add=True)

                @pl.when(c == 0)
                def _():
                    pltpu.sync_copy(wsplat, accd.at[dstidx.at[0]], add=True)

                return 0

            lax.fori_loop(0, NBLK, bbody, 0)
        plsc.subcore_barrier()

        out_off = c * NPAD + row0
        pltpu.sync_copy(acc.at[pl.ds(row0, ROWS_PT)],
                        outf_hbm.at[pl.ds(out_off, ROWS_PT)])

        @pl.when(c == 0)
        def _():
            pltpu.sync_copy(accd.at[pl.ds(row0, ROWS_PT)],
                            outd_hbm.at[pl.ds(row0, ROWS_PT)])

        if rel == 0:
            zero_acc()


def _sc_edge(scal, feat_a2, feat_b2, src_a, dst_a, src_b, dst_b):
    f32 = jnp.float32
    mesh = plsc.VectorSubcoreMesh(core_axis_name="c", subcore_axis_name="s",
                                  num_cores=NC, num_subcores=NS)
    kern = pl.kernel(
        _sc_body,
        out_type=[
            jax.ShapeDtypeStruct((NC * NPAD, 128), f32),
            jax.ShapeDtypeStruct((NPAD, DEN_W), f32),
            jax.ShapeDtypeStruct((NC * NPAD, 128), f32),
            jax.ShapeDtypeStruct((NPAD, DEN_W), f32),
        ],
        mesh=mesh,
        compiler_params=pltpu.CompilerParams(use_tc_tiling_on_sc=False,
                                             needs_layout_passes=False),
        scratch_types=[
            pltpu.VMEM((_NS1,), f32),
            pltpu.VMEM((_NS1,), f32),
            pltpu.VMEM((SEG,), jnp.int32),
            pltpu.VMEM((SEG,), jnp.int32),
            pltpu.VMEM((BLK,), f32),
            pltpu.VMEM((BLK,), jnp.int32),
            pltpu.VMEM((BLK, 128), f32),
            pltpu.VMEM((BLK, DEN_W), f32),
            pltpu.VMEM((2, BLK), jnp.int32),
            pltpu.SemaphoreType.DMA,
            pltpu.VMEM_SHARED((NPAD, 128), f32),
            pltpu.VMEM_SHARED((NPAD, DEN_W), f32),
        ],
    )
    return kern(scal, feat_a2, feat_b2, src_a, dst_a, src_b, dst_b)


_BN3 = 1024
_G3 = NPAD // _BN3


def _s3a_body(accA_ref, denA_ref, accB_ref, denB_ref,
              biasA_ref, biasB_ref, w1_ref, b1_ref, w2_ref,
              z2_ref, ps_ref):
    i = pl.program_id(0)

    @pl.when(i == 0)
    def _():
        ps_ref[...] = jnp.zeros((8, 128), jnp.float32)

    rowid = i * _BN3 + lax.broadcasted_iota(jnp.int32, (_BN3, 1), 0)
    valid = rowid < N
    add = jnp.zeros((8, 128), jnp.float32)
    for r, (acc_ref, den_ref, bias_ref) in enumerate(
            ((accA_ref, denA_ref, biasA_ref), (accB_ref, denB_ref, biasB_ref))):
        blk = acc_ref[...]
        feat = jnp.concatenate([blk[0], blk[1]], axis=1)
        denom = den_ref[...][:, 0:1]
        safe = jnp.where(denom == 0, 1.0, denom)
        pre = feat / safe + bias_ref[...][None, :]
        z = jnp.where(pre > 0, pre, jnp.exp(jnp.minimum(pre, 0.0)) - 1.0)
        z2_ref[r] = z
        t = jnp.tanh(jnp.dot(z, w1_ref[...], preferred_element_type=jnp.float32)
                     + b1_ref[...][None, :])
        sv = jnp.dot(t, w2_ref[...], preferred_element_type=jnp.float32)
        partial = jnp.sum(jnp.where(valid, sv, 0.0))
        onehot = ((lax.broadcasted_iota(jnp.int32, (8, 128), 0) == r)
                  & (lax.broadcasted_iota(jnp.int32, (8, 128), 1) == i))
        add = add + jnp.where(onehot, partial, 0.0)
    ps_ref[...] += add


def _s3b_body(z2_ref, ps_ref, out_ref):
    m = jnp.sum(ps_ref[...], axis=1) / float(N)
    mx = jnp.maximum(m[0], m[1])
    a0 = jnp.exp(m[0] - mx)
    a1 = jnp.exp(m[1] - mx)
    tot = a0 + a1
    out_ref[...] = (a0 / tot) * z2_ref[0] + (a1 / tot) * z2_ref[1]


def _stage3(accA, denA, accB, denB, biasA, biasB, w1, b1, w2):
    f32 = jnp.float32
    full = lambda shape: pl.BlockSpec(shape, lambda i: tuple(0 for _ in shape))
    z2, ps = pl.pallas_call(
        _s3a_body,
        grid=(_G3,),
        in_specs=[
            pl.BlockSpec((2, _BN3, 128), lambda i: (0, i, 0)),
            pl.BlockSpec((_BN3, DEN_W), lambda i: (i, 0)),
            pl.BlockSpec((2, _BN3, 128), lambda i: (0, i, 0)),
            pl.BlockSpec((_BN3, DEN_W), lambda i: (i, 0)),
            full((D,)), full((D,)),
            full((D, H_SEM)), full((H_SEM,)), full((H_SEM, 1)),
        ],
        out_specs=[
            pl.BlockSpec((2, _BN3, D), lambda i: (0, i, 0)),
            pl.BlockSpec((8, 128), lambda i: (0, 0)),
        ],
        out_shape=[
            jax.ShapeDtypeStruct((2, NPAD, D), f32),
            jax.ShapeDtypeStruct((8, 128), f32),
        ],
    )(accA, denA, accB, denB, biasA, biasB, w1, b1, w2)
    out = pl.pallas_call(
        _s3b_body,
        grid=(_G3,),
        in_specs=[
            pl.BlockSpec((2, _BN3, D), lambda i: (0, i, 0)),
            full((8, 128)),
        ],
        out_specs=pl.BlockSpec((_BN3, D), lambda i: (i, 0)),
        out_shape=jax.ShapeDtypeStruct((NPAD, D), f32),
    )(z2, ps)
    return out


def kernel(dst_feat, src_feat_a, src_feat_b, edge_index_a, edge_index_b,
           W_T_dst, b_T_dst, W_T_a, b_T_a, gatA_W, gatA_attn_l, gatA_attn_r,
           gatA_bias, gatB_W, gatB_attn_l, gatB_attn_r, gatB_bias,
           sem_W1, sem_b1, sem_W2):
    feat_a2, feat_b2, scal = _stage1(
        dst_feat, src_feat_a, src_feat_b, W_T_dst, b_T_dst, W_T_a, b_T_a,
        gatA_W, gatA_attn_l, gatA_attn_r, gatB_W, gatB_attn_l, gatB_attn_r)
    outf_a, outd_a, outf_b, outd_b = _sc_edge(
        scal, feat_a2.reshape(2 * N, 128), feat_b2.reshape(2 * N, 128),
        edge_index_a[0], edge_index_a[1], edge_index_b[0], edge_index_b[1])
    out = _stage3(outf_a.reshape(NC, NPAD, 128), outd_a,
                  outf_b.reshape(NC, NPAD, 128), outd_b,
                  gatA_bias, gatB_bias, sem_W1, sem_b1, sem_W2)
    return out[:N]
